# baseline (speedup 1.0000x reference)
import functools

import jax
import jax.numpy as jnp
from jax import lax
from jax.experimental import pallas as pl
from jax.experimental.pallas import tpu as pltpu
from jax.experimental.pallas import tpu_sc as plsc

B, LD, LQ, V, D, NF = 4096, 200, 20, 1000000, 64, 4
W = NF + D
NC, NS = 2, 16
NW = NC * NS
CH = 512
KI = CH // 128
R1 = B * LD
R2 = B * LQ
Q1 = R1 // CH // NW
Q2 = R2 // CH // NW

_mesh = plsc.VectorSubcoreMesh(core_axis_name="c", subcore_axis_name="s")


def _gather_body(idx_hbm, tbl_hbm, out_hbm, idx_v, row_v, sem0, sem1, nq):
    wid = lax.axis_index("s") * NC + lax.axis_index("c")
    sems = (sem0, sem1)

    def fire(r0, p):
        pltpu.sync_copy(idx_hbm.at[pl.ds(r0, CH)], idx_v.at[p])
        for j in range(KI):
            pltpu.async_copy(
                tbl_hbm.at[idx_v.at[p, pl.ds(j * 128, 128)]],
                row_v.at[p, pl.ds(j * 128, 128)],
                sems[p],
            )

    def finish(q, p):
        for j in range(KI):
            pltpu.make_async_copy(
                tbl_hbm.at[idx_v.at[p, pl.ds(j * 128, 128)]],
                row_v.at[p, pl.ds(j * 128, 128)],
                sems[p],
            ).wait()
        pltpu.sync_copy(row_v.at[p], out_hbm.at[pl.ds(q * CH, CH)])

    q0 = wid * nq
    fire(q0 * CH, 0)

    def pair(t, carry):
        q = q0 + 2 * t
        fire((q + 1) * CH, 1)
        finish(q, 0)

        @pl.when(q + 2 < q0 + nq)
        def _():
            fire((q + 2) * CH, 0)

        finish(q + 1, 1)
        return carry

    lax.fori_loop(0, nq // 2, pair, 0)

    @pl.when(nq % 2 == 1)
    def _():
        finish(q0 + nq - 1, 0)


def _sc_gather(nrows, nq):
    return functools.partial(
        pl.kernel,
        mesh=_mesh,
        compiler_params=pltpu.CompilerParams(use_tc_tiling_on_sc=False,
                                             needs_layout_passes=False),
        out_type=jax.ShapeDtypeStruct((nrows, D), jnp.float32),
        scratch_types=[
            pltpu.VMEM((2, CH), jnp.int32),
            pltpu.VMEM((2, CH, D), jnp.float32),
            pltpu.SemaphoreType.DMA,
            pltpu.SemaphoreType.DMA,
        ],
    )(functools.partial(_gather_body, nq=nq))


_gather_x1 = _sc_gather(R1, Q1)
_gather_x2 = _sc_gather(R2, Q2)


def kernel(x1, x1_f, x1_pos, x1_ner, x1_mask, x2, x2_mask, sent_lens, emb_table):
    del x1_pos, x1_ner, x1_mask, x2_mask, sent_lens
    e1 = _gather_x1(x1.reshape(R1), emb_table)
    e2 = _gather_x2(x2.reshape(R2), emb_table)
    x1_all = (jnp.pad(x1_f, ((0, 0), (0, 0), (0, D))) +
              jnp.pad(e1.reshape(B, LD, D), ((0, 0), (0, 0), (NF, 0))))
    return x1_all, e2.reshape(B, LQ, D)
